# Initial kernel scaffold; baseline (speedup 1.0000x reference)
import functools

import jax
import jax.numpy as jnp
from jax import lax
from jax.experimental import pallas as pl
from jax.experimental.pallas import tpu as pltpu
from jax.experimental.pallas import tpu_sc as plsc

N = 10000
E = 320000
H = 128
R = 16
CUTOFF = 5.0
NCLS = 128

NW = 32
EPW = E // NW
CS = 80
NCHUNK = EPW // CS
NPW = N // 16

_SC_MESH = dict(core_axis_name="c", subcore_axis_name="s")


def _zc_body(z_hbm, col_hbm, zc_hbm, col_v, zc_v, sem):
    wid = lax.axis_index("s") * 2 + lax.axis_index("c")
    base = wid * EPW

    def chunk(t, carry):
        off = base + t * CS
        pltpu.sync_copy(col_hbm.at[pl.ds(off, CS)], col_v)
        pltpu.async_copy(z_hbm.at[col_v], zc_v, sem).wait()
        pltpu.sync_copy(zc_v, zc_hbm.at[pl.ds(off, CS)])
        return carry

    lax.fori_loop(0, NCHUNK, chunk, 0)


_zc_kernel = functools.partial(
    pl.kernel,
    out_type=jax.ShapeDtypeStruct((E,), jnp.int32),
    mesh=plsc.VectorSubcoreMesh(**_SC_MESH),
    scratch_types=[
        pltpu.VMEM((CS,), jnp.int32),
        pltpu.VMEM((CS,), jnp.int32),
        pltpu.SemaphoreType.DMA,
    ],
)(_zc_body)


def _msg_body(ea_ref, dist_ref, zc_ref, wdt_ref, bd_ref, ntab_ref, msg_ref):
    d = dist_ref[...]
    c = 0.5 * (jnp.cos(jnp.pi * d / CUTOFF) + 1.0)
    c = c * (d < CUTOFF).astype(jnp.float32)
    w = (
        jnp.dot(ea_ref[...], wdt_ref[...], precision=lax.Precision.HIGHEST)
        + bd_ref[...]
    )
    oh = (zc_ref[...] == lax.broadcasted_iota(jnp.int32, (1, NCLS), 1)).astype(
        jnp.float32
    )
    nf = jnp.dot(oh, ntab_ref[...], precision=lax.Precision.HIGHEST)
    msg_ref[...] = w * c * nf


def _msg_call(ea, dist, zc, wdt, bd2, ntab_pad, block_e):
    grid = (E // block_e,)
    return pl.pallas_call(
        _msg_body,
        grid=grid,
        in_specs=[
            pl.BlockSpec((block_e, R), lambda i: (i, 0)),
            pl.BlockSpec((block_e, 1), lambda i: (i, 0)),
            pl.BlockSpec((block_e, 1), lambda i: (i, 0)),
            pl.BlockSpec((R, H), lambda i: (0, 0)),
            pl.BlockSpec((1, H), lambda i: (0, 0)),
            pl.BlockSpec((NCLS, H), lambda i: (0, 0)),
        ],
        out_specs=pl.BlockSpec((block_e, H), lambda i: (i, 0)),
        out_shape=jax.ShapeDtypeStruct((E, H), jnp.float32),
    )(ea, dist, zc, wdt, bd2, ntab_pad)


def _scatter_body(msg_hbm, row_hbm, zinit_hbm, parts_hbm, row_v, msg_v, agg_sp):
    cid = lax.axis_index("c")
    sid = lax.axis_index("s")
    wid = sid * 2 + cid
    base = wid * EPW

    nbase = sid * NPW
    pltpu.sync_copy(zinit_hbm.at[pl.ds(nbase, NPW)], agg_sp.at[pl.ds(nbase, NPW)])
    plsc.subcore_barrier()

    def chunk(t, carry):
        off = base + t * CS
        pltpu.sync_copy(row_hbm.at[pl.ds(off, CS)], row_v)
        pltpu.sync_copy(msg_hbm.at[pl.ds(off, CS)], msg_v)
        pltpu.sync_copy(msg_v, agg_sp.at[row_v], add=True)
        return carry

    lax.fori_loop(0, NCHUNK, chunk, 0)
    plsc.subcore_barrier()
    pltpu.sync_copy(
        agg_sp.at[pl.ds(nbase, NPW)], parts_hbm.at[cid, pl.ds(nbase, NPW)]
    )


_scatter_kernel = functools.partial(
    pl.kernel,
    out_type=jax.ShapeDtypeStruct((2, N, H), jnp.float32),
    mesh=plsc.VectorSubcoreMesh(**_SC_MESH),
    scratch_types=[
        pltpu.VMEM((CS,), jnp.int32),
        pltpu.VMEM((CS, H), jnp.float32),
        pltpu.VMEM_SHARED((N, H), jnp.float32),
    ],
)(_scatter_body)


def _out_body(z_ref, p0_ref, p1_ref, atab_ref, wcat_ref, wcbt_ref, bc_ref, out_ref):
    oh = (z_ref[...] == lax.broadcasted_iota(jnp.int32, (1, NCLS), 1)).astype(
        jnp.float32
    )
    t1 = jnp.dot(atab_ref[...], wcat_ref[...], precision=lax.Precision.HIGHEST)
    agg = p0_ref[...] + p1_ref[...]
    out_ref[...] = (
        jnp.dot(oh, t1, precision=lax.Precision.HIGHEST)
        + jnp.dot(agg, wcbt_ref[...], precision=lax.Precision.HIGHEST)
        + bc_ref[...]
    )


def _out_call(z2, p0, p1, atab_pad, wcat, wcbt, bc2, block_n):
    grid = (N // block_n,)
    return pl.pallas_call(
        _out_body,
        grid=grid,
        in_specs=[
            pl.BlockSpec((block_n, 1), lambda i: (i, 0)),
            pl.BlockSpec((block_n, H), lambda i: (i, 0)),
            pl.BlockSpec((block_n, H), lambda i: (i, 0)),
            pl.BlockSpec((NCLS, H), lambda i: (0, 0)),
            pl.BlockSpec((H, H), lambda i: (0, 0)),
            pl.BlockSpec((H, H), lambda i: (0, 0)),
            pl.BlockSpec((1, H), lambda i: (0, 0)),
        ],
        out_specs=pl.BlockSpec((block_n, H), lambda i: (i, 0)),
        out_shape=jax.ShapeDtypeStruct((N, H), jnp.float32),
    )(z2, p0, p1, atab_pad, wcat, wcbt, bc2)


def kernel(z, edge_index, edge_dist, edge_attr, atom_table, neigh_table, Wd, bd, Wc, bc):
    z = z.astype(jnp.int32)
    row = edge_index[0].astype(jnp.int32)
    col = edge_index[1].astype(jnp.int32)

    zc = _zc_kernel(z, col)

    dist2 = edge_dist.reshape(E, 1)
    zc2 = zc.reshape(E, 1)
    wdt = Wd.T
    bd2 = bd.reshape(1, H)
    ntab_pad = jnp.zeros((NCLS, H), jnp.float32).at[: neigh_table.shape[0]].set(neigh_table)
    msg = _msg_call(edge_attr, dist2, zc2, wdt, bd2, ntab_pad, block_e=2000)

    zinit = jnp.zeros((N, H), jnp.float32)
    parts = _scatter_kernel(msg, row, zinit)

    z2 = z.reshape(N, 1)
    atab_pad = jnp.zeros((NCLS, H), jnp.float32).at[: atom_table.shape[0]].set(atom_table)
    wcat = Wc[:, :H].T
    wcbt = Wc[:, H:].T
    bc2 = bc.reshape(1, H)
    return _out_call(z2, parts[0], parts[1], atab_pad, wcat, wcbt, bc2, block_n=1000)


# speedup vs baseline: 1.3928x; 1.3928x over previous
import functools

import jax
import jax.numpy as jnp
from jax import lax
from jax.experimental import pallas as pl
from jax.experimental.pallas import tpu as pltpu
from jax.experimental.pallas import tpu_sc as plsc

N = 10000
E = 320000
H = 128
R = 16
CUTOFF = 5.0
NCLS = 128

NW = 32
EPW = E // NW
CS = 80
NCHUNK = EPW // CS
NPW = 624
NPW_LAST = N - 15 * NPW

_SC_MESH = dict(core_axis_name="c", subcore_axis_name="s")


def _zc_body(z_hbm, col_hbm, zc_hbm, col_v, zc_v, sem):
    wid = lax.axis_index("s") * 2 + lax.axis_index("c")
    base = wid * EPW

    def chunk(t, carry):
        off = base + t * CS
        pltpu.sync_copy(col_hbm.at[pl.ds(off, CS)], col_v)
        pltpu.async_copy(z_hbm.at[col_v], zc_v, sem).wait()
        pltpu.sync_copy(zc_v, zc_hbm.at[pl.ds(off, CS)])
        return carry

    lax.fori_loop(0, NCHUNK, chunk, 0)


@functools.cache
def _zc_kernel():
    return pl.kernel(
        _zc_body,
        out_type=jax.ShapeDtypeStruct((E,), jnp.int32),
        mesh=plsc.VectorSubcoreMesh(**_SC_MESH),
        scratch_types=[
            pltpu.VMEM((CS,), jnp.int32),
            pltpu.VMEM((CS,), jnp.int32),
            pltpu.SemaphoreType.DMA,
        ],
    )


def _msg_body(ea_ref, dist_ref, zc_ref, wdt_ref, bd_ref, ntab_ref, msg_ref):
    d = dist_ref[...]
    c = 0.5 * (jnp.cos(jnp.pi * d / CUTOFF) + 1.0)
    c = c * (d < CUTOFF).astype(jnp.float32)
    w = (
        jnp.dot(ea_ref[...], wdt_ref[...], precision=lax.Precision.HIGHEST)
        + bd_ref[...]
    )
    oh = (zc_ref[...] == lax.broadcasted_iota(jnp.int32, (1, NCLS), 1)).astype(
        jnp.float32
    )
    nf = jnp.dot(oh, ntab_ref[...], precision=lax.Precision.HIGHEST)
    msg_ref[...] = w * c * nf


def _msg_call(ea, dist, zc, wdt, bd2, ntab_pad, block_e):
    grid = (E // block_e,)
    return pl.pallas_call(
        _msg_body,
        grid=grid,
        in_specs=[
            pl.BlockSpec((block_e, R), lambda i: (i, 0)),
            pl.BlockSpec((block_e, 1), lambda i: (i, 0)),
            pl.BlockSpec((block_e, 1), lambda i: (i, 0)),
            pl.BlockSpec((R, H), lambda i: (0, 0)),
            pl.BlockSpec((1, H), lambda i: (0, 0)),
            pl.BlockSpec((NCLS, H), lambda i: (0, 0)),
        ],
        out_specs=pl.BlockSpec((block_e, H), lambda i: (i, 0)),
        out_shape=jax.ShapeDtypeStruct((E, H), jnp.float32),
    )(ea, dist, zc, wdt, bd2, ntab_pad)


def _scatter_body(msg_hbm, row_hbm, zinit_hbm, parts_hbm, row_v, msg_v, agg_sp):
    cid = lax.axis_index("c")
    sid = lax.axis_index("s")
    wid = sid * 2 + cid
    base = wid * EPW

    nbase = sid * NPW

    @pl.when(sid < 15)
    def _():
        pltpu.sync_copy(
            zinit_hbm.at[pl.ds(nbase, NPW)], agg_sp.at[pl.ds(nbase, NPW)]
        )

    @pl.when(sid == 15)
    def _():
        pltpu.sync_copy(
            zinit_hbm.at[pl.ds(nbase, NPW_LAST)], agg_sp.at[pl.ds(nbase, NPW_LAST)]
        )

    plsc.subcore_barrier()

    def chunk(t, carry):
        off = base + t * CS
        pltpu.sync_copy(row_hbm.at[pl.ds(off, CS)], row_v)
        pltpu.sync_copy(msg_hbm.at[pl.ds(off, CS)], msg_v)
        pltpu.sync_copy(msg_v, agg_sp.at[row_v], add=True)
        return carry

    lax.fori_loop(0, NCHUNK, chunk, 0)
    plsc.subcore_barrier()

    @pl.when(sid < 15)
    def _():
        pltpu.sync_copy(
            agg_sp.at[pl.ds(nbase, NPW)], parts_hbm.at[cid, pl.ds(nbase, NPW)]
        )

    @pl.when(sid == 15)
    def _():
        pltpu.sync_copy(
            agg_sp.at[pl.ds(nbase, NPW_LAST)],
            parts_hbm.at[cid, pl.ds(nbase, NPW_LAST)],
        )


@functools.cache
def _scatter_kernel():
    return pl.kernel(
        _scatter_body,
        out_type=jax.ShapeDtypeStruct((2, N, H), jnp.float32),
        mesh=plsc.VectorSubcoreMesh(**_SC_MESH),
        scratch_types=[
            pltpu.VMEM((CS,), jnp.int32),
            pltpu.VMEM((CS, H), jnp.float32),
            pltpu.VMEM_SHARED((N, H), jnp.float32),
        ],
    )


def _out_body(z_ref, p0_ref, p1_ref, atab_ref, wcat_ref, wcbt_ref, bc_ref, out_ref):
    oh = (z_ref[...] == lax.broadcasted_iota(jnp.int32, (1, NCLS), 1)).astype(
        jnp.float32
    )
    t1 = jnp.dot(atab_ref[...], wcat_ref[...], precision=lax.Precision.HIGHEST)
    agg = p0_ref[...] + p1_ref[...]
    out_ref[...] = (
        jnp.dot(oh, t1, precision=lax.Precision.HIGHEST)
        + jnp.dot(agg, wcbt_ref[...], precision=lax.Precision.HIGHEST)
        + bc_ref[...]
    )


def _out_call(z2, p0, p1, atab_pad, wcat, wcbt, bc2, block_n):
    grid = (N // block_n,)
    return pl.pallas_call(
        _out_body,
        grid=grid,
        in_specs=[
            pl.BlockSpec((block_n, 1), lambda i: (i, 0)),
            pl.BlockSpec((block_n, H), lambda i: (i, 0)),
            pl.BlockSpec((block_n, H), lambda i: (i, 0)),
            pl.BlockSpec((NCLS, H), lambda i: (0, 0)),
            pl.BlockSpec((H, H), lambda i: (0, 0)),
            pl.BlockSpec((H, H), lambda i: (0, 0)),
            pl.BlockSpec((1, H), lambda i: (0, 0)),
        ],
        out_specs=pl.BlockSpec((block_n, H), lambda i: (i, 0)),
        out_shape=jax.ShapeDtypeStruct((N, H), jnp.float32),
    )(z2, p0, p1, atab_pad, wcat, wcbt, bc2)


def kernel(z, edge_index, edge_dist, edge_attr, atom_table, neigh_table, Wd, bd, Wc, bc):
    z = z.astype(jnp.int32)
    row = edge_index[0].astype(jnp.int32)
    col = edge_index[1].astype(jnp.int32)

    zc = _zc_kernel()(z, col)

    dist2 = edge_dist.reshape(E, 1)
    zc2 = zc.reshape(E, 1)
    wdt = Wd.T
    bd2 = bd.reshape(1, H)
    ntab_pad = jnp.zeros((NCLS, H), jnp.float32).at[: neigh_table.shape[0]].set(neigh_table)
    msg = _msg_call(edge_attr, dist2, zc2, wdt, bd2, ntab_pad, block_e=2000)

    zinit = jnp.zeros((N, H), jnp.float32)
    parts = _scatter_kernel()(msg, row, zinit)

    z2 = z.reshape(N, 1)
    atab_pad = jnp.zeros((NCLS, H), jnp.float32).at[: atom_table.shape[0]].set(atom_table)
    wcat = Wc[:, :H].T
    wcbt = Wc[:, H:].T
    bc2 = bc.reshape(1, H)
    return _out_call(z2, parts[0], parts[1], atab_pad, wcat, wcbt, bc2, block_n=1000)


# speedup vs baseline: 1.6507x; 1.1852x over previous
import functools

import jax
import jax.numpy as jnp
from jax import lax
from jax.experimental import pallas as pl
from jax.experimental.pallas import tpu as pltpu
from jax.experimental.pallas import tpu_sc as plsc

N = 10000
E = 320000
H = 128
R = 16
CUTOFF = 5.0
NCLS = 128

NW = 32
EPW = E // NW
CS = 80
NCHUNK = EPW // CS
NPW = 624
NPW_LAST = N - 15 * NPW

_SC_MESH = dict(core_axis_name="c", subcore_axis_name="s")


def _zc_body(z_hbm, col_hbm, zc_hbm, col_v, zc_v, sem):
    wid = lax.axis_index("s") * 2 + lax.axis_index("c")
    base = wid * EPW

    def chunk(t, carry):
        off = base + t * CS
        pltpu.sync_copy(col_hbm.at[pl.ds(off, CS)], col_v)
        pltpu.async_copy(z_hbm.at[col_v], zc_v, sem).wait()
        pltpu.sync_copy(zc_v, zc_hbm.at[pl.ds(off, CS)])
        return carry

    lax.fori_loop(0, NCHUNK, chunk, 0)


@functools.cache
def _zc_kernel():
    return pl.kernel(
        _zc_body,
        out_type=jax.ShapeDtypeStruct((E,), jnp.int32),
        mesh=plsc.VectorSubcoreMesh(**_SC_MESH),
        scratch_types=[
            pltpu.VMEM((CS,), jnp.int32),
            pltpu.VMEM((CS,), jnp.int32),
            pltpu.SemaphoreType.DMA,
        ],
    )


def _msg_body(ea_ref, dist_ref, zc_ref, wdt_ref, bd_ref, ntab_ref, msg_ref):
    be = dist_ref.shape[0]
    d = dist_ref[...].reshape(be, 1)
    c = 0.5 * (jnp.cos(jnp.pi * d / CUTOFF) + 1.0)
    c = c * (d < CUTOFF).astype(jnp.float32)
    w = (
        jnp.dot(ea_ref[...], wdt_ref[...], precision=lax.Precision.HIGHEST)
        + bd_ref[...]
    )
    zc2 = zc_ref[...].reshape(be, 1)
    oh = (zc2 == lax.broadcasted_iota(jnp.int32, (1, NCLS), 1)).astype(jnp.float32)
    nf = jnp.dot(oh, ntab_ref[...])
    msg_ref[...] = w * nf * c


def _msg_call(ea, dist, zc, wdt, bd2, ntab_pad, block_e):
    grid = (E // block_e,)
    return pl.pallas_call(
        _msg_body,
        grid=grid,
        in_specs=[
            pl.BlockSpec((block_e, R), lambda i: (i, 0)),
            pl.BlockSpec((block_e,), lambda i: (i,)),
            pl.BlockSpec((block_e,), lambda i: (i,)),
            pl.BlockSpec((R, H), lambda i: (0, 0)),
            pl.BlockSpec((1, H), lambda i: (0, 0)),
            pl.BlockSpec((NCLS, H), lambda i: (0, 0)),
        ],
        out_specs=pl.BlockSpec((block_e, H), lambda i: (i, 0)),
        out_shape=jax.ShapeDtypeStruct((E, H), jnp.float32),
    )(ea, dist, zc, wdt, bd2, ntab_pad)


def _scatter_body(msg_hbm, row_hbm, zinit_hbm, parts_hbm, row_v, msg_v, agg_sp):
    cid = lax.axis_index("c")
    sid = lax.axis_index("s")
    wid = sid * 2 + cid
    base = wid * EPW

    nbase = sid * NPW

    @pl.when(sid < 15)
    def _():
        pltpu.sync_copy(
            zinit_hbm.at[pl.ds(nbase, NPW)], agg_sp.at[pl.ds(nbase, NPW)]
        )

    @pl.when(sid == 15)
    def _():
        pltpu.sync_copy(
            zinit_hbm.at[pl.ds(nbase, NPW_LAST)], agg_sp.at[pl.ds(nbase, NPW_LAST)]
        )

    plsc.subcore_barrier()

    def chunk(t, carry):
        off = base + t * CS
        pltpu.sync_copy(row_hbm.at[pl.ds(off, CS)], row_v)
        pltpu.sync_copy(msg_hbm.at[pl.ds(off, CS)], msg_v)
        pltpu.sync_copy(msg_v, agg_sp.at[row_v], add=True)
        return carry

    lax.fori_loop(0, NCHUNK, chunk, 0)
    plsc.subcore_barrier()

    @pl.when(sid < 15)
    def _():
        pltpu.sync_copy(
            agg_sp.at[pl.ds(nbase, NPW)], parts_hbm.at[cid, pl.ds(nbase, NPW)]
        )

    @pl.when(sid == 15)
    def _():
        pltpu.sync_copy(
            agg_sp.at[pl.ds(nbase, NPW_LAST)],
            parts_hbm.at[cid, pl.ds(nbase, NPW_LAST)],
        )


@functools.cache
def _scatter_kernel():
    return pl.kernel(
        _scatter_body,
        out_type=jax.ShapeDtypeStruct((2, N, H), jnp.float32),
        mesh=plsc.VectorSubcoreMesh(**_SC_MESH),
        scratch_types=[
            pltpu.VMEM((CS,), jnp.int32),
            pltpu.VMEM((CS, H), jnp.float32),
            pltpu.VMEM_SHARED((N, H), jnp.float32),
        ],
    )


def _out_body(z_ref, p0_ref, p1_ref, atab_ref, wcat_ref, wcbt_ref, bc_ref, out_ref):
    oh = (z_ref[...] == lax.broadcasted_iota(jnp.int32, (1, NCLS), 1)).astype(
        jnp.float32
    )
    t1 = jnp.dot(atab_ref[...], wcat_ref[...], precision=lax.Precision.HIGHEST)
    agg = p0_ref[...] + p1_ref[...]
    out_ref[...] = (
        jnp.dot(oh, t1, precision=lax.Precision.HIGHEST)
        + jnp.dot(agg, wcbt_ref[...], precision=lax.Precision.HIGHEST)
        + bc_ref[...]
    )


def _out_call(z2, p0, p1, atab_pad, wcat, wcbt, bc2, block_n):
    grid = (N // block_n,)
    return pl.pallas_call(
        _out_body,
        grid=grid,
        in_specs=[
            pl.BlockSpec((block_n, 1), lambda i: (i, 0)),
            pl.BlockSpec((block_n, H), lambda i: (i, 0)),
            pl.BlockSpec((block_n, H), lambda i: (i, 0)),
            pl.BlockSpec((NCLS, H), lambda i: (0, 0)),
            pl.BlockSpec((H, H), lambda i: (0, 0)),
            pl.BlockSpec((H, H), lambda i: (0, 0)),
            pl.BlockSpec((1, H), lambda i: (0, 0)),
        ],
        out_specs=pl.BlockSpec((block_n, H), lambda i: (i, 0)),
        out_shape=jax.ShapeDtypeStruct((N, H), jnp.float32),
    )(z2, p0, p1, atab_pad, wcat, wcbt, bc2)


def kernel(z, edge_index, edge_dist, edge_attr, atom_table, neigh_table, Wd, bd, Wc, bc):
    z = z.astype(jnp.int32)
    row = edge_index[0].astype(jnp.int32)
    col = edge_index[1].astype(jnp.int32)

    zc = _zc_kernel()(z, col)

    wdt = Wd.T
    bd2 = bd.reshape(1, H)
    ntab_pad = jnp.zeros((NCLS, H), jnp.float32).at[: neigh_table.shape[0]].set(neigh_table)
    msg = _msg_call(edge_attr, edge_dist, zc, wdt, bd2, ntab_pad, block_e=512)

    zinit = jnp.zeros((N, H), jnp.float32)
    parts = _scatter_kernel()(msg, row, zinit)

    z2 = z.reshape(N, 1)
    atab_pad = jnp.zeros((NCLS, H), jnp.float32).at[: atom_table.shape[0]].set(atom_table)
    wcat = Wc[:, :H].T
    wcbt = Wc[:, H:].T
    bc2 = bc.reshape(1, H)
    return _out_call(z2, parts[0], parts[1], atab_pad, wcat, wcbt, bc2, block_n=1000)


# speedup vs baseline: 2.1133x; 1.2802x over previous
import functools

import jax
import jax.numpy as jnp
from jax import lax
from jax.experimental import pallas as pl
from jax.experimental.pallas import tpu as pltpu
from jax.experimental.pallas import tpu_sc as plsc

N = 10000
E = 320000
H = 128
R = 16
CUTOFF = 5.0
NCLS = 128

NW = 32
EPW = E // NW
CS = 80
NCHUNK = EPW // CS
NPW = 624
NPW_LAST = N - 15 * NPW

_SC_MESH = dict(core_axis_name="c", subcore_axis_name="s")


def _zc_body(z_hbm, col_hbm, zc_hbm, col_v, zc_v, sem):
    wid = lax.axis_index("s") * 2 + lax.axis_index("c")
    base = wid * EPW

    def chunk(t, carry):
        off = base + t * CS
        pltpu.sync_copy(col_hbm.at[pl.ds(off, CS)], col_v)
        pltpu.async_copy(z_hbm.at[col_v], zc_v, sem).wait()
        pltpu.sync_copy(zc_v, zc_hbm.at[pl.ds(off, CS)])
        return carry

    lax.fori_loop(0, NCHUNK, chunk, 0)


@functools.cache
def _zc_kernel():
    return pl.kernel(
        _zc_body,
        out_type=jax.ShapeDtypeStruct((E,), jnp.int32),
        mesh=plsc.VectorSubcoreMesh(**_SC_MESH),
        scratch_types=[
            pltpu.VMEM((CS,), jnp.int32),
            pltpu.VMEM((CS,), jnp.int32),
            pltpu.SemaphoreType.DMA,
        ],
    )


def _msg_body(ea_ref, dist_ref, zc_ref, wdt_ref, bd_ref, ntab_ref, msg_ref):
    be = dist_ref.shape[0]
    d = dist_ref[...]
    c = 0.5 * (jnp.cos(jnp.pi * d / CUTOFF) + 1.0)
    c = c * (d < CUTOFF).astype(jnp.float32)
    w = (
        jnp.dot(ea_ref[...], wdt_ref[...], precision=lax.Precision.HIGHEST)
        + bd_ref[...]
    )
    ohct = (
        lax.broadcasted_iota(jnp.int32, (NCLS, be), 0) == zc_ref[...][None, :]
    ).astype(jnp.float32) * c[None, :]
    nfc = lax.dot_general(
        ohct, ntab_ref[...], (((0,), (0,)), ((), ())),
    )
    msg_ref[...] = w * nfc


def _msg_call(ea, dist, zc, wdt, bd2, ntab_pad, block_e):
    grid = (E // block_e,)
    return pl.pallas_call(
        _msg_body,
        grid=grid,
        in_specs=[
            pl.BlockSpec((block_e, R), lambda i: (i, 0)),
            pl.BlockSpec((block_e,), lambda i: (i,)),
            pl.BlockSpec((block_e,), lambda i: (i,)),
            pl.BlockSpec((R, H), lambda i: (0, 0)),
            pl.BlockSpec((1, H), lambda i: (0, 0)),
            pl.BlockSpec((NCLS, H), lambda i: (0, 0)),
        ],
        out_specs=pl.BlockSpec((block_e, H), lambda i: (i, 0)),
        out_shape=jax.ShapeDtypeStruct((E, H), jnp.float32),
    )(ea, dist, zc, wdt, bd2, ntab_pad)


def _scatter_body(msg_hbm, row_hbm, zinit_hbm, parts_hbm, row_v, msg_v, agg_sp):
    cid = lax.axis_index("c")
    sid = lax.axis_index("s")
    wid = sid * 2 + cid
    base = wid * EPW

    nbase = sid * NPW

    @pl.when(sid < 15)
    def _():
        pltpu.sync_copy(
            zinit_hbm.at[pl.ds(nbase, NPW)], agg_sp.at[pl.ds(nbase, NPW)]
        )

    @pl.when(sid == 15)
    def _():
        pltpu.sync_copy(
            zinit_hbm.at[pl.ds(nbase, NPW_LAST)], agg_sp.at[pl.ds(nbase, NPW_LAST)]
        )

    plsc.subcore_barrier()

    def chunk(t, carry):
        off = base + t * CS
        pltpu.sync_copy(row_hbm.at[pl.ds(off, CS)], row_v)
        pltpu.sync_copy(msg_hbm.at[pl.ds(off, CS)], msg_v)
        pltpu.sync_copy(msg_v, agg_sp.at[row_v], add=True)
        return carry

    lax.fori_loop(0, NCHUNK, chunk, 0)
    plsc.subcore_barrier()

    @pl.when(sid < 15)
    def _():
        pltpu.sync_copy(
            agg_sp.at[pl.ds(nbase, NPW)], parts_hbm.at[cid, pl.ds(nbase, NPW)]
        )

    @pl.when(sid == 15)
    def _():
        pltpu.sync_copy(
            agg_sp.at[pl.ds(nbase, NPW_LAST)],
            parts_hbm.at[cid, pl.ds(nbase, NPW_LAST)],
        )


@functools.cache
def _scatter_kernel():
    return pl.kernel(
        _scatter_body,
        out_type=jax.ShapeDtypeStruct((2, N, H), jnp.float32),
        mesh=plsc.VectorSubcoreMesh(**_SC_MESH),
        scratch_types=[
            pltpu.VMEM((CS,), jnp.int32),
            pltpu.VMEM((CS, H), jnp.float32),
            pltpu.VMEM_SHARED((N, H), jnp.float32),
        ],
    )


def _out_body(z_ref, p0_ref, p1_ref, atab_ref, wcat_ref, wcbt_ref, bc_ref, out_ref):
    oh = (z_ref[...] == lax.broadcasted_iota(jnp.int32, (1, NCLS), 1)).astype(
        jnp.float32
    )
    t1 = jnp.dot(atab_ref[...], wcat_ref[...], precision=lax.Precision.HIGHEST)
    agg = p0_ref[...] + p1_ref[...]
    out_ref[...] = (
        jnp.dot(oh, t1, precision=lax.Precision.HIGHEST)
        + jnp.dot(agg, wcbt_ref[...], precision=lax.Precision.HIGHEST)
        + bc_ref[...]
    )


def _out_call(z2, p0, p1, atab_pad, wcat, wcbt, bc2, block_n):
    grid = (N // block_n,)
    return pl.pallas_call(
        _out_body,
        grid=grid,
        in_specs=[
            pl.BlockSpec((block_n, 1), lambda i: (i, 0)),
            pl.BlockSpec((block_n, H), lambda i: (i, 0)),
            pl.BlockSpec((block_n, H), lambda i: (i, 0)),
            pl.BlockSpec((NCLS, H), lambda i: (0, 0)),
            pl.BlockSpec((H, H), lambda i: (0, 0)),
            pl.BlockSpec((H, H), lambda i: (0, 0)),
            pl.BlockSpec((1, H), lambda i: (0, 0)),
        ],
        out_specs=pl.BlockSpec((block_n, H), lambda i: (i, 0)),
        out_shape=jax.ShapeDtypeStruct((N, H), jnp.float32),
    )(z2, p0, p1, atab_pad, wcat, wcbt, bc2)


def kernel(z, edge_index, edge_dist, edge_attr, atom_table, neigh_table, Wd, bd, Wc, bc):
    z = z.astype(jnp.int32)
    row = edge_index[0].astype(jnp.int32)
    col = edge_index[1].astype(jnp.int32)

    zc = _zc_kernel()(z, col)

    wdt = Wd.T
    bd2 = bd.reshape(1, H)
    ntab_pad = jnp.zeros((NCLS, H), jnp.float32).at[: neigh_table.shape[0]].set(neigh_table)
    msg = _msg_call(edge_attr, edge_dist, zc, wdt, bd2, ntab_pad, block_e=512)

    zinit = jnp.zeros((N, H), jnp.float32)
    parts = _scatter_kernel()(msg, row, zinit)

    z2 = z.reshape(N, 1)
    atab_pad = jnp.zeros((NCLS, H), jnp.float32).at[: atom_table.shape[0]].set(atom_table)
    wcat = Wc[:, :H].T
    wcbt = Wc[:, H:].T
    bc2 = bc.reshape(1, H)
    return _out_call(z2, parts[0], parts[1], atab_pad, wcat, wcbt, bc2, block_n=1000)


# speedup vs baseline: 2.5522x; 1.2077x over previous
import functools

import jax
import jax.numpy as jnp
from jax import lax
from jax.experimental import pallas as pl
from jax.experimental.pallas import tpu as pltpu
from jax.experimental.pallas import tpu_sc as plsc

N = 10000
E = 320000
H = 128
R = 16
CUTOFF = 5.0
NCLS = 128

NW = 32
EPW = E // NW
CS = 80
NCHUNK = EPW // CS
NPW = 624
NPW_LAST = N - 15 * NPW

_SC_MESH = dict(core_axis_name="c", subcore_axis_name="s")


ZG = 5


def _zc_body(z_hbm, col2_hbm, zc_hbm, col_v, zc_v, sem):
    wid = lax.axis_index("s") * 2 + lax.axis_index("c")

    pltpu.sync_copy(col2_hbm.at[wid], col_v)

    def group(g, carry):
        for j in range(ZG):
            sl = pl.ds((g * ZG + j) * CS, CS)
            pltpu.async_copy(z_hbm.at[col_v.at[sl]], zc_v.at[sl], sem)
        for j in range(ZG):
            pltpu.make_async_copy(
                z_hbm.at[col_v.at[pl.ds(0, CS)]], zc_v.at[pl.ds(0, CS)], sem
            ).wait()
        return carry

    lax.fori_loop(0, NCHUNK // ZG, group, 0)
    pltpu.sync_copy(zc_v, zc_hbm.at[wid])


@functools.cache
def _zc_kernel():
    return pl.kernel(
        _zc_body,
        out_type=jax.ShapeDtypeStruct((NW, EPW), jnp.int32),
        mesh=plsc.VectorSubcoreMesh(**_SC_MESH),
        scratch_types=[
            pltpu.VMEM((EPW,), jnp.int32),
            pltpu.VMEM((EPW,), jnp.int32),
            pltpu.SemaphoreType.DMA,
        ],
    )


def _msg_body(ea_ref, dist_ref, zc_ref, wdt_ref, bd_ref, ntab_ref, msg_ref):
    be = dist_ref.shape[0]
    d = dist_ref[...]
    c = 0.5 * (jnp.cos(jnp.pi * d / CUTOFF) + 1.0)
    c = c * (d < CUTOFF).astype(jnp.float32)
    w = (
        jnp.dot(ea_ref[...], wdt_ref[...], precision=lax.Precision.HIGHEST)
        + bd_ref[...]
    )
    ohct = (
        lax.broadcasted_iota(jnp.int32, (NCLS, be), 0) == zc_ref[...][None, :]
    ).astype(jnp.float32) * c[None, :]
    nfc = lax.dot_general(
        ohct, ntab_ref[...], (((0,), (0,)), ((), ())),
    )
    msg_ref[...] = w * nfc


def _msg_call(ea, dist, zc, wdt, bd2, ntab_pad, block_e):
    grid = (E // block_e,)
    return pl.pallas_call(
        _msg_body,
        grid=grid,
        in_specs=[
            pl.BlockSpec((block_e, R), lambda i: (i, 0)),
            pl.BlockSpec((block_e,), lambda i: (i,)),
            pl.BlockSpec((block_e,), lambda i: (i,)),
            pl.BlockSpec((R, H), lambda i: (0, 0)),
            pl.BlockSpec((1, H), lambda i: (0, 0)),
            pl.BlockSpec((NCLS, H), lambda i: (0, 0)),
        ],
        out_specs=pl.BlockSpec((block_e, H), lambda i: (i, 0)),
        out_shape=jax.ShapeDtypeStruct((E, H), jnp.float32),
    )(ea, dist, zc, wdt, bd2, ntab_pad)


MCS = CS
NMC = EPW // MCS


def _scatter_body(
    msg_hbm, row3_hbm, zinit_hbm, parts_hbm, row_v, msg_a, msg_b, agg_sp, sem_a, sem_b
):
    cid = lax.axis_index("c")
    sid = lax.axis_index("s")
    wid = sid * 2 + cid
    base = wid * EPW

    nbase = sid * NPW

    @pl.when(sid < 15)
    def _():
        pltpu.sync_copy(
            zinit_hbm.at[pl.ds(nbase, NPW)], agg_sp.at[pl.ds(nbase, NPW)]
        )

    @pl.when(sid == 15)
    def _():
        pltpu.sync_copy(
            zinit_hbm.at[pl.ds(nbase, NPW_LAST)], agg_sp.at[pl.ds(nbase, NPW_LAST)]
        )

    pltpu.sync_copy(row3_hbm.at[wid], row_v)
    plsc.subcore_barrier()

    pltpu.async_copy(msg_hbm.at[pl.ds(base, MCS)], msg_a, sem_a)

    def scatter_from(buf, t):
        pltpu.sync_copy(buf, agg_sp.at[row_v.at[t]], add=True)

    def pair(p, carry):
        t0 = p * 2
        pltpu.make_async_copy(msg_hbm.at[pl.ds(base, MCS)], msg_a, sem_a).wait()

        @pl.when(t0 + 1 < NMC)
        def _():
            pltpu.async_copy(
                msg_hbm.at[pl.ds(base + (t0 + 1) * MCS, MCS)], msg_b, sem_b
            )

        scatter_from(msg_a, t0)

        @pl.when(t0 + 1 < NMC)
        def _():
            pltpu.make_async_copy(
                msg_hbm.at[pl.ds(base, MCS)], msg_b, sem_b
            ).wait()

            @pl.when(t0 + 2 < NMC)
            def _():
                pltpu.async_copy(
                    msg_hbm.at[pl.ds(base + (t0 + 2) * MCS, MCS)], msg_a, sem_a
                )

            scatter_from(msg_b, t0 + 1)

        return carry

    lax.fori_loop(0, (NMC + 1) // 2, pair, 0)
    plsc.subcore_barrier()

    @pl.when(sid < 15)
    def _():
        pltpu.sync_copy(
            agg_sp.at[pl.ds(nbase, NPW)], parts_hbm.at[cid, pl.ds(nbase, NPW)]
        )

    @pl.when(sid == 15)
    def _():
        pltpu.sync_copy(
            agg_sp.at[pl.ds(nbase, NPW_LAST)],
            parts_hbm.at[cid, pl.ds(nbase, NPW_LAST)],
        )


@functools.cache
def _scatter_kernel():
    return pl.kernel(
        _scatter_body,
        out_type=jax.ShapeDtypeStruct((2, N, H), jnp.float32),
        mesh=plsc.VectorSubcoreMesh(**_SC_MESH),
        scratch_types=[
            pltpu.VMEM((NCHUNK, CS), jnp.int32),
            pltpu.VMEM((MCS, H), jnp.float32),
            pltpu.VMEM((MCS, H), jnp.float32),
            pltpu.VMEM_SHARED((N, H), jnp.float32),
            pltpu.SemaphoreType.DMA,
            pltpu.SemaphoreType.DMA,
        ],
    )


def _out_body(z_ref, p0_ref, p1_ref, atab_ref, wcat_ref, wcbt_ref, bc_ref, out_ref):
    oh = (z_ref[...] == lax.broadcasted_iota(jnp.int32, (1, NCLS), 1)).astype(
        jnp.float32
    )
    t1 = jnp.dot(atab_ref[...], wcat_ref[...], precision=lax.Precision.HIGHEST)
    agg = p0_ref[...] + p1_ref[...]
    out_ref[...] = (
        jnp.dot(oh, t1, precision=lax.Precision.HIGHEST)
        + jnp.dot(agg, wcbt_ref[...], precision=lax.Precision.HIGHEST)
        + bc_ref[...]
    )


def _out_call(z2, p0, p1, atab_pad, wcat, wcbt, bc2, block_n):
    grid = (N // block_n,)
    return pl.pallas_call(
        _out_body,
        grid=grid,
        in_specs=[
            pl.BlockSpec((block_n, 1), lambda i: (i, 0)),
            pl.BlockSpec((block_n, H), lambda i: (i, 0)),
            pl.BlockSpec((block_n, H), lambda i: (i, 0)),
            pl.BlockSpec((NCLS, H), lambda i: (0, 0)),
            pl.BlockSpec((H, H), lambda i: (0, 0)),
            pl.BlockSpec((H, H), lambda i: (0, 0)),
            pl.BlockSpec((1, H), lambda i: (0, 0)),
        ],
        out_specs=pl.BlockSpec((block_n, H), lambda i: (i, 0)),
        out_shape=jax.ShapeDtypeStruct((N, H), jnp.float32),
    )(z2, p0, p1, atab_pad, wcat, wcbt, bc2)


def kernel(z, edge_index, edge_dist, edge_attr, atom_table, neigh_table, Wd, bd, Wc, bc):
    z = z.astype(jnp.int32)
    row = edge_index[0].astype(jnp.int32)
    col = edge_index[1].astype(jnp.int32)

    zc = _zc_kernel()(z, col.reshape(NW, EPW)).reshape(E)

    wdt = Wd.T
    bd2 = bd.reshape(1, H)
    ntab_pad = jnp.zeros((NCLS, H), jnp.float32).at[: neigh_table.shape[0]].set(neigh_table)
    msg = _msg_call(edge_attr, edge_dist, zc, wdt, bd2, ntab_pad, block_e=512)

    zinit = jnp.zeros((N, H), jnp.float32)
    parts = _scatter_kernel()(msg, row.reshape(NW, NCHUNK, CS), zinit)

    z2 = z.reshape(N, 1)
    atab_pad = jnp.zeros((NCLS, H), jnp.float32).at[: atom_table.shape[0]].set(atom_table)
    wcat = Wc[:, :H].T
    wcbt = Wc[:, H:].T
    bc2 = bc.reshape(1, H)
    return _out_call(z2, parts[0], parts[1], atab_pad, wcat, wcbt, bc2, block_n=1000)


# speedup vs baseline: 3.0363x; 1.1897x over previous
import functools

import jax
import jax.numpy as jnp
from jax import lax
from jax.experimental import pallas as pl
from jax.experimental.pallas import tpu as pltpu
from jax.experimental.pallas import tpu_sc as plsc

N = 10000
E = 320000
H = 128
R = 16
CUTOFF = 5.0
NCLS = 128

NW = 32
EPW = E // NW
CS = 80
NCHUNK = EPW // CS
NPW = 624
NPW_LAST = N - 15 * NPW

_SC_MESH = dict(core_axis_name="c", subcore_axis_name="s")


ZG = 5


def _zc_body(z_hbm, col2_hbm, zc_hbm, col_v, zc_v, sem):
    wid = lax.axis_index("s") * 2 + lax.axis_index("c")

    pltpu.sync_copy(col2_hbm.at[wid], col_v)

    def group(g, carry):
        for j in range(ZG):
            sl = pl.ds((g * ZG + j) * CS, CS)
            pltpu.async_copy(z_hbm.at[col_v.at[sl]], zc_v.at[sl], sem)
        for j in range(ZG):
            pltpu.make_async_copy(
                z_hbm.at[col_v.at[pl.ds(0, CS)]], zc_v.at[pl.ds(0, CS)], sem
            ).wait()
        return carry

    lax.fori_loop(0, NCHUNK // ZG, group, 0)
    pltpu.sync_copy(zc_v, zc_hbm.at[wid])


@functools.cache
def _zc_kernel():
    return pl.kernel(
        _zc_body,
        out_type=jax.ShapeDtypeStruct((NW, EPW), jnp.int32),
        mesh=plsc.VectorSubcoreMesh(**_SC_MESH),
        scratch_types=[
            pltpu.VMEM((EPW,), jnp.int32),
            pltpu.VMEM((EPW,), jnp.int32),
            pltpu.SemaphoreType.DMA,
        ],
    )


def _msg_body(eat_ref, dist_ref, zc_ref, wdt_ref, bd_ref, ntab_ref, msg_ref):
    be = dist_ref.shape[0]
    d = dist_ref[...]
    c = 0.5 * (jnp.cos(jnp.pi * d / CUTOFF) + 1.0)
    c = c * (d < CUTOFF).astype(jnp.float32)
    w = (
        lax.dot_general(eat_ref[...], wdt_ref[...], (((0,), (0,)), ((), ())))
        + bd_ref[...]
    )
    ohct = (
        lax.broadcasted_iota(jnp.int32, (NCLS, be), 0) == zc_ref[...][None, :]
    ).astype(jnp.float32) * c[None, :]
    nfc = lax.dot_general(
        ohct, ntab_ref[...], (((0,), (0,)), ((), ())),
    )
    msg_ref[...] = w * nfc


def _msg_call(eat, dist, zc, wdt, bd2, ntab_pad, block_e):
    grid = (E // block_e,)
    return pl.pallas_call(
        _msg_body,
        grid=grid,
        in_specs=[
            pl.BlockSpec((R, block_e), lambda i: (0, i)),
            pl.BlockSpec((block_e,), lambda i: (i,)),
            pl.BlockSpec((block_e,), lambda i: (i,)),
            pl.BlockSpec((R, H), lambda i: (0, 0)),
            pl.BlockSpec((1, H), lambda i: (0, 0)),
            pl.BlockSpec((NCLS, H), lambda i: (0, 0)),
        ],
        out_specs=pl.BlockSpec((block_e, H), lambda i: (i, 0)),
        out_shape=jax.ShapeDtypeStruct((E, H), jnp.float32),
    )(eat, dist, zc, wdt, bd2, ntab_pad)


MCS = CS
NMC = EPW // MCS


def _scatter_body(
    msg_hbm, row3_hbm, zinit_hbm, parts_hbm, row_v, msg_a, msg_b, agg_sp, sem_a, sem_b
):
    cid = lax.axis_index("c")
    sid = lax.axis_index("s")
    wid = sid * 2 + cid
    base = wid * EPW

    nbase = sid * NPW

    @pl.when(sid < 15)
    def _():
        pltpu.sync_copy(
            zinit_hbm.at[pl.ds(nbase, NPW)], agg_sp.at[pl.ds(nbase, NPW)]
        )

    @pl.when(sid == 15)
    def _():
        pltpu.sync_copy(
            zinit_hbm.at[pl.ds(nbase, NPW_LAST)], agg_sp.at[pl.ds(nbase, NPW_LAST)]
        )

    pltpu.sync_copy(row3_hbm.at[wid], row_v)
    plsc.subcore_barrier()

    pltpu.async_copy(msg_hbm.at[pl.ds(base, MCS)], msg_a, sem_a)

    def scatter_from(buf, t):
        pltpu.sync_copy(buf, agg_sp.at[row_v.at[t]], add=True)

    def pair(p, carry):
        t0 = p * 2
        pltpu.make_async_copy(msg_hbm.at[pl.ds(base, MCS)], msg_a, sem_a).wait()

        @pl.when(t0 + 1 < NMC)
        def _():
            pltpu.async_copy(
                msg_hbm.at[pl.ds(base + (t0 + 1) * MCS, MCS)], msg_b, sem_b
            )

        scatter_from(msg_a, t0)

        @pl.when(t0 + 1 < NMC)
        def _():
            pltpu.make_async_copy(
                msg_hbm.at[pl.ds(base, MCS)], msg_b, sem_b
            ).wait()

            @pl.when(t0 + 2 < NMC)
            def _():
                pltpu.async_copy(
                    msg_hbm.at[pl.ds(base + (t0 + 2) * MCS, MCS)], msg_a, sem_a
                )

            scatter_from(msg_b, t0 + 1)

        return carry

    lax.fori_loop(0, (NMC + 1) // 2, pair, 0)
    plsc.subcore_barrier()

    @pl.when(sid < 15)
    def _():
        pltpu.sync_copy(
            agg_sp.at[pl.ds(nbase, NPW)], parts_hbm.at[cid, pl.ds(nbase, NPW)]
        )

    @pl.when(sid == 15)
    def _():
        pltpu.sync_copy(
            agg_sp.at[pl.ds(nbase, NPW_LAST)],
            parts_hbm.at[cid, pl.ds(nbase, NPW_LAST)],
        )


@functools.cache
def _scatter_kernel():
    return pl.kernel(
        _scatter_body,
        out_type=jax.ShapeDtypeStruct((2, N, H), jnp.float32),
        mesh=plsc.VectorSubcoreMesh(**_SC_MESH),
        scratch_types=[
            pltpu.VMEM((NCHUNK, CS), jnp.int32),
            pltpu.VMEM((MCS, H), jnp.float32),
            pltpu.VMEM((MCS, H), jnp.float32),
            pltpu.VMEM_SHARED((N, H), jnp.float32),
            pltpu.SemaphoreType.DMA,
            pltpu.SemaphoreType.DMA,
        ],
    )


def _out_body(z_ref, p0_ref, p1_ref, atab_ref, wcat_ref, wcbt_ref, bc_ref, out_ref):
    oh = (z_ref[...] == lax.broadcasted_iota(jnp.int32, (1, NCLS), 1)).astype(
        jnp.float32
    )
    t1 = jnp.dot(atab_ref[...], wcat_ref[...], precision=lax.Precision.HIGHEST)
    agg = p0_ref[...] + p1_ref[...]
    out_ref[...] = (
        jnp.dot(oh, t1, precision=lax.Precision.HIGHEST)
        + jnp.dot(agg, wcbt_ref[...], precision=lax.Precision.HIGHEST)
        + bc_ref[...]
    )


def _out_call(z2, p0, p1, atab_pad, wcat, wcbt, bc2, block_n):
    grid = (N // block_n,)
    return pl.pallas_call(
        _out_body,
        grid=grid,
        in_specs=[
            pl.BlockSpec((block_n, 1), lambda i: (i, 0)),
            pl.BlockSpec((block_n, H), lambda i: (i, 0)),
            pl.BlockSpec((block_n, H), lambda i: (i, 0)),
            pl.BlockSpec((NCLS, H), lambda i: (0, 0)),
            pl.BlockSpec((H, H), lambda i: (0, 0)),
            pl.BlockSpec((H, H), lambda i: (0, 0)),
            pl.BlockSpec((1, H), lambda i: (0, 0)),
        ],
        out_specs=pl.BlockSpec((block_n, H), lambda i: (i, 0)),
        out_shape=jax.ShapeDtypeStruct((N, H), jnp.float32),
    )(z2, p0, p1, atab_pad, wcat, wcbt, bc2)


def kernel(z, edge_index, edge_dist, edge_attr, atom_table, neigh_table, Wd, bd, Wc, bc):
    z = z.astype(jnp.int32)
    row = edge_index[0].astype(jnp.int32)
    col = edge_index[1].astype(jnp.int32)

    zc = _zc_kernel()(z, col.reshape(NW, EPW)).reshape(E)

    wdt = Wd.T
    bd2 = bd.reshape(1, H)
    ntab_pad = jnp.zeros((NCLS, H), jnp.float32).at[: neigh_table.shape[0]].set(neigh_table)
    msg = _msg_call(edge_attr.T, edge_dist, zc, wdt, bd2, ntab_pad, block_e=512)

    zinit = jnp.zeros((N, H), jnp.float32)
    parts = _scatter_kernel()(msg, row.reshape(NW, NCHUNK, CS), zinit)

    z2 = z.reshape(N, 1)
    atab_pad = jnp.zeros((NCLS, H), jnp.float32).at[: atom_table.shape[0]].set(atom_table)
    wcat = Wc[:, :H].T
    wcbt = Wc[:, H:].T
    bc2 = bc.reshape(1, H)
    return _out_call(z2, parts[0], parts[1], atab_pad, wcat, wcbt, bc2, block_n=1000)


# speedup vs baseline: 5.2134x; 1.7171x over previous
import functools

import jax
import jax.numpy as jnp
from jax import lax
from jax.experimental import pallas as pl
from jax.experimental.pallas import tpu as pltpu
from jax.experimental.pallas import tpu_sc as plsc

N = 10000
E = 320000
H = 128
R = 16
CUTOFF = 5.0
NCLS = 128

NW = 32
EPW = E // NW
CS = 80
NCHUNK = EPW // CS
NPW = 624
NPW_LAST = N - 15 * NPW

_SC_MESH = dict(core_axis_name="c", subcore_axis_name="s")


ZG = 5


def _zc_body(z_hbm, col2_hbm, zc_hbm, col_v, zc_v, sem):
    wid = lax.axis_index("s") * 2 + lax.axis_index("c")

    pltpu.sync_copy(col2_hbm.at[wid], col_v)

    def group(g, carry):
        for j in range(ZG):
            sl = pl.ds((g * ZG + j) * CS, CS)
            pltpu.async_copy(z_hbm.at[col_v.at[sl]], zc_v.at[sl], sem)
        for j in range(ZG):
            pltpu.make_async_copy(
                z_hbm.at[col_v.at[pl.ds(0, CS)]], zc_v.at[pl.ds(0, CS)], sem
            ).wait()
        return carry

    lax.fori_loop(0, NCHUNK // ZG, group, 0)
    pltpu.sync_copy(zc_v, zc_hbm.at[wid])


@functools.cache
def _zc_kernel():
    return pl.kernel(
        _zc_body,
        out_type=jax.ShapeDtypeStruct((NW, EPW), jnp.int32),
        mesh=plsc.VectorSubcoreMesh(**_SC_MESH),
        scratch_types=[
            pltpu.VMEM((EPW,), jnp.int32),
            pltpu.VMEM((EPW,), jnp.int32),
            pltpu.SemaphoreType.DMA,
        ],
    )


MB_GRP = 512


def _msg_body(eat_ref, dist_ref, zc_ref, wdt_ref, bd_ref, ntab_ref, msg_ref):
    be = eat_ref.shape[1]
    for j in range(be // MB_GRP):
        sl = pl.ds(j * MB_GRP, MB_GRP)
        d = dist_ref[0, 0, sl]
        c = 0.5 * (jnp.cos(jnp.pi * d / CUTOFF) + 1.0)
        c = c * (d < CUTOFF).astype(jnp.float32)
        w = (
            lax.dot_general(
                eat_ref[:, sl], wdt_ref[...], (((0,), (0,)), ((), ()))
            )
            + bd_ref[...]
        )
        ohct = (
            lax.broadcasted_iota(jnp.int32, (NCLS, MB_GRP), 0)
            == zc_ref[0, 0, sl][None, :]
        ).astype(jnp.float32) * c[None, :]
        nfc = lax.dot_general(ohct, ntab_ref[...], (((0,), (0,)), ((), ())))
        msg_ref[sl, :] = w * nfc


def _msg_call(eat, dist, zc, wdt, bd2, ntab_pad, block_e):
    grid = (E // block_e,)
    dist3 = dist.reshape(E // block_e, 1, block_e)
    zc3 = zc.reshape(E // block_e, 1, block_e)
    return pl.pallas_call(
        _msg_body,
        grid=grid,
        in_specs=[
            pl.BlockSpec((R, block_e), lambda i: (0, i)),
            pl.BlockSpec((1, 1, block_e), lambda i: (i, 0, 0)),
            pl.BlockSpec((1, 1, block_e), lambda i: (i, 0, 0)),
            pl.BlockSpec((R, H), lambda i: (0, 0)),
            pl.BlockSpec((1, H), lambda i: (0, 0)),
            pl.BlockSpec((NCLS, H), lambda i: (0, 0)),
        ],
        out_specs=pl.BlockSpec((block_e, H), lambda i: (i, 0)),
        out_shape=jax.ShapeDtypeStruct((E, H), jnp.float32),
        compiler_params=pltpu.CompilerParams(
            dimension_semantics=("parallel",),
        ),
    )(eat, dist3, zc3, wdt, bd2, ntab_pad)


MCS = CS
NMC = EPW // MCS


def _scatter_body(
    msg_hbm, row3_hbm, zinit_hbm, parts_hbm, row_v, msg_a, msg_b, agg_sp, sem_a, sem_b
):
    cid = lax.axis_index("c")
    sid = lax.axis_index("s")
    wid = sid * 2 + cid
    base = wid * EPW

    nbase = sid * NPW

    @pl.when(sid < 15)
    def _():
        pltpu.sync_copy(
            zinit_hbm.at[pl.ds(nbase, NPW)], agg_sp.at[pl.ds(nbase, NPW)]
        )

    @pl.when(sid == 15)
    def _():
        pltpu.sync_copy(
            zinit_hbm.at[pl.ds(nbase, NPW_LAST)], agg_sp.at[pl.ds(nbase, NPW_LAST)]
        )

    pltpu.sync_copy(row3_hbm.at[wid], row_v)
    plsc.subcore_barrier()

    pltpu.async_copy(msg_hbm.at[pl.ds(base, MCS)], msg_a, sem_a)

    def scatter_from(buf, t):
        pltpu.sync_copy(buf, agg_sp.at[row_v.at[t]], add=True)

    def pair(p, carry):
        t0 = p * 2
        pltpu.make_async_copy(msg_hbm.at[pl.ds(base, MCS)], msg_a, sem_a).wait()

        @pl.when(t0 + 1 < NMC)
        def _():
            pltpu.async_copy(
                msg_hbm.at[pl.ds(base + (t0 + 1) * MCS, MCS)], msg_b, sem_b
            )

        scatter_from(msg_a, t0)

        @pl.when(t0 + 1 < NMC)
        def _():
            pltpu.make_async_copy(
                msg_hbm.at[pl.ds(base, MCS)], msg_b, sem_b
            ).wait()

            @pl.when(t0 + 2 < NMC)
            def _():
                pltpu.async_copy(
                    msg_hbm.at[pl.ds(base + (t0 + 2) * MCS, MCS)], msg_a, sem_a
                )

            scatter_from(msg_b, t0 + 1)

        return carry

    lax.fori_loop(0, (NMC + 1) // 2, pair, 0)
    plsc.subcore_barrier()

    @pl.when(sid < 15)
    def _():
        pltpu.sync_copy(
            agg_sp.at[pl.ds(nbase, NPW)], parts_hbm.at[cid, pl.ds(nbase, NPW)]
        )

    @pl.when(sid == 15)
    def _():
        pltpu.sync_copy(
            agg_sp.at[pl.ds(nbase, NPW_LAST)],
            parts_hbm.at[cid, pl.ds(nbase, NPW_LAST)],
        )


@functools.cache
def _scatter_kernel():
    return pl.kernel(
        _scatter_body,
        out_type=jax.ShapeDtypeStruct((2, N, H), jnp.float32),
        mesh=plsc.VectorSubcoreMesh(**_SC_MESH),
        scratch_types=[
            pltpu.VMEM((NCHUNK, CS), jnp.int32),
            pltpu.VMEM((MCS, H), jnp.float32),
            pltpu.VMEM((MCS, H), jnp.float32),
            pltpu.VMEM_SHARED((N, H), jnp.float32),
            pltpu.SemaphoreType.DMA,
            pltpu.SemaphoreType.DMA,
        ],
    )


def _out_body(z_ref, p0_ref, p1_ref, atab_ref, wcat_ref, wcbt_ref, bc_ref, out_ref):
    oh = (z_ref[...] == lax.broadcasted_iota(jnp.int32, (1, NCLS), 1)).astype(
        jnp.float32
    )
    t1 = jnp.dot(atab_ref[...], wcat_ref[...], precision=lax.Precision.HIGHEST)
    agg = p0_ref[...] + p1_ref[...]
    out_ref[...] = (
        jnp.dot(oh, t1, precision=lax.Precision.HIGHEST)
        + jnp.dot(agg, wcbt_ref[...], precision=lax.Precision.HIGHEST)
        + bc_ref[...]
    )


def _out_call(z2, p0, p1, atab_pad, wcat, wcbt, bc2, block_n):
    grid = (N // block_n,)
    return pl.pallas_call(
        _out_body,
        grid=grid,
        in_specs=[
            pl.BlockSpec((block_n, 1), lambda i: (i, 0)),
            pl.BlockSpec((block_n, H), lambda i: (i, 0)),
            pl.BlockSpec((block_n, H), lambda i: (i, 0)),
            pl.BlockSpec((NCLS, H), lambda i: (0, 0)),
            pl.BlockSpec((H, H), lambda i: (0, 0)),
            pl.BlockSpec((H, H), lambda i: (0, 0)),
            pl.BlockSpec((1, H), lambda i: (0, 0)),
        ],
        out_specs=pl.BlockSpec((block_n, H), lambda i: (i, 0)),
        out_shape=jax.ShapeDtypeStruct((N, H), jnp.float32),
    )(z2, p0, p1, atab_pad, wcat, wcbt, bc2)


def kernel(z, edge_index, edge_dist, edge_attr, atom_table, neigh_table, Wd, bd, Wc, bc):
    z = z.astype(jnp.int32)
    row = edge_index[0].astype(jnp.int32)
    col = edge_index[1].astype(jnp.int32)

    zc = _zc_kernel()(z, col.reshape(NW, EPW)).reshape(E)

    wdt = Wd.T
    bd2 = bd.reshape(1, H)
    ntab_pad = jnp.zeros((NCLS, H), jnp.float32).at[: neigh_table.shape[0]].set(neigh_table)
    msg = _msg_call(edge_attr.T, edge_dist, zc, wdt, bd2, ntab_pad, block_e=2560)

    zinit = jnp.zeros((N, H), jnp.float32)
    parts = _scatter_kernel()(msg, row.reshape(NW, NCHUNK, CS), zinit)

    z2 = z.reshape(N, 1)
    atab_pad = jnp.zeros((NCLS, H), jnp.float32).at[: atom_table.shape[0]].set(atom_table)
    wcat = Wc[:, :H].T
    wcbt = Wc[:, H:].T
    bc2 = bc.reshape(1, H)
    return _out_call(z2, parts[0], parts[1], atab_pad, wcat, wcbt, bc2, block_n=1000)


# speedup vs baseline: 5.3097x; 1.0185x over previous
import functools

import jax
import jax.numpy as jnp
from jax import lax
from jax.experimental import pallas as pl
from jax.experimental.pallas import tpu as pltpu
from jax.experimental.pallas import tpu_sc as plsc

N = 10000
E = 320000
H = 128
R = 16
CUTOFF = 5.0
NCLS = 128

NW = 32
EPW = E // NW
CS = 80
NCHUNK = EPW // CS
NPW = 624
NPW_LAST = N - 15 * NPW

_SC_MESH = dict(core_axis_name="c", subcore_axis_name="s")


ZG = 5


def _zc_body(z_hbm, col2_hbm, zc_hbm, col_v, zc_v, sem):
    wid = lax.axis_index("s") * 2 + lax.axis_index("c")

    pltpu.sync_copy(col2_hbm.at[wid], col_v)

    def fire(g):
        for j in range(ZG):
            sl = pl.ds((g * ZG + j) * CS, CS)
            pltpu.async_copy(z_hbm.at[col_v.at[sl]], zc_v.at[sl], sem)

    def drain():
        for j in range(ZG):
            pltpu.make_async_copy(
                z_hbm.at[col_v.at[pl.ds(0, CS)]], zc_v.at[pl.ds(0, CS)], sem
            ).wait()

    ngrp = NCHUNK // ZG
    fire(0)

    def group(g, carry):
        @pl.when(g + 1 < ngrp)
        def _():
            fire(g + 1)

        drain()
        return carry

    lax.fori_loop(0, ngrp, group, 0)
    pltpu.sync_copy(zc_v, zc_hbm.at[wid])


@functools.cache
def _zc_kernel():
    return pl.kernel(
        _zc_body,
        out_type=jax.ShapeDtypeStruct((NW, EPW), jnp.int32),
        mesh=plsc.VectorSubcoreMesh(**_SC_MESH),
        scratch_types=[
            pltpu.VMEM((EPW,), jnp.int32),
            pltpu.VMEM((EPW,), jnp.int32),
            pltpu.SemaphoreType.DMA,
        ],
    )


MB_GRP = 512


def _msg_body(eat_ref, dist_ref, zc_ref, wdt_ref, bd_ref, ntab_ref, msg_ref):
    be = eat_ref.shape[1]
    for j in range(be // MB_GRP):
        sl = pl.ds(j * MB_GRP, MB_GRP)
        d = dist_ref[0, 0, sl]
        c = 0.5 * (jnp.cos(jnp.pi * d / CUTOFF) + 1.0)
        c = c * (d < CUTOFF).astype(jnp.float32)
        w = (
            lax.dot_general(
                eat_ref[:, sl], wdt_ref[...], (((0,), (0,)), ((), ()))
            )
            + bd_ref[...]
        )
        ohct = (
            lax.broadcasted_iota(jnp.int32, (NCLS, MB_GRP), 0)
            == zc_ref[0, 0, sl][None, :]
        ).astype(jnp.float32) * c[None, :]
        nfc = lax.dot_general(ohct, ntab_ref[...], (((0,), (0,)), ((), ())))
        msg_ref[sl, :] = w * nfc


def _msg_call(eat, dist, zc, wdt, bd2, ntab_pad, block_e):
    grid = (E // block_e,)
    dist3 = dist.reshape(E // block_e, 1, block_e)
    zc3 = zc.reshape(E // block_e, 1, block_e)
    return pl.pallas_call(
        _msg_body,
        grid=grid,
        in_specs=[
            pl.BlockSpec((R, block_e), lambda i: (0, i)),
            pl.BlockSpec((1, 1, block_e), lambda i: (i, 0, 0)),
            pl.BlockSpec((1, 1, block_e), lambda i: (i, 0, 0)),
            pl.BlockSpec((R, H), lambda i: (0, 0)),
            pl.BlockSpec((1, H), lambda i: (0, 0)),
            pl.BlockSpec((NCLS, H), lambda i: (0, 0)),
        ],
        out_specs=pl.BlockSpec((block_e, H), lambda i: (i, 0)),
        out_shape=jax.ShapeDtypeStruct((E, H), jnp.float32),
        compiler_params=pltpu.CompilerParams(
            dimension_semantics=("parallel",),
        ),
    )(eat, dist3, zc3, wdt, bd2, ntab_pad)


MCS = CS
NMC = EPW // MCS


def _scatter_body(
    msg_hbm, row3_hbm, zinit_hbm, parts_hbm, row_v, msg_a, msg_b, agg_sp, sem_a, sem_b
):
    cid = lax.axis_index("c")
    sid = lax.axis_index("s")
    wid = sid * 2 + cid
    base = wid * EPW

    nbase = sid * NPW

    @pl.when(sid < 15)
    def _():
        pltpu.sync_copy(
            zinit_hbm.at[pl.ds(nbase, NPW)], agg_sp.at[pl.ds(nbase, NPW)]
        )

    @pl.when(sid == 15)
    def _():
        pltpu.sync_copy(
            zinit_hbm.at[pl.ds(nbase, NPW_LAST)], agg_sp.at[pl.ds(nbase, NPW_LAST)]
        )

    pltpu.sync_copy(row3_hbm.at[wid], row_v)
    plsc.subcore_barrier()

    pltpu.async_copy(msg_hbm.at[pl.ds(base, MCS)], msg_a, sem_a)

    def scatter_from(buf, t):
        pltpu.sync_copy(buf, agg_sp.at[row_v.at[t]], add=True)

    def pair(p, carry):
        t0 = p * 2
        pltpu.make_async_copy(msg_hbm.at[pl.ds(base, MCS)], msg_a, sem_a).wait()

        @pl.when(t0 + 1 < NMC)
        def _():
            pltpu.async_copy(
                msg_hbm.at[pl.ds(base + (t0 + 1) * MCS, MCS)], msg_b, sem_b
            )

        scatter_from(msg_a, t0)

        @pl.when(t0 + 1 < NMC)
        def _():
            pltpu.make_async_copy(
                msg_hbm.at[pl.ds(base, MCS)], msg_b, sem_b
            ).wait()

            @pl.when(t0 + 2 < NMC)
            def _():
                pltpu.async_copy(
                    msg_hbm.at[pl.ds(base + (t0 + 2) * MCS, MCS)], msg_a, sem_a
                )

            scatter_from(msg_b, t0 + 1)

        return carry

    lax.fori_loop(0, (NMC + 1) // 2, pair, 0)
    plsc.subcore_barrier()

    @pl.when(sid < 15)
    def _():
        pltpu.sync_copy(
            agg_sp.at[pl.ds(nbase, NPW)], parts_hbm.at[cid, pl.ds(nbase, NPW)]
        )

    @pl.when(sid == 15)
    def _():
        pltpu.sync_copy(
            agg_sp.at[pl.ds(nbase, NPW_LAST)],
            parts_hbm.at[cid, pl.ds(nbase, NPW_LAST)],
        )


@functools.cache
def _scatter_kernel():
    return pl.kernel(
        _scatter_body,
        out_type=jax.ShapeDtypeStruct((2, N, H), jnp.float32),
        mesh=plsc.VectorSubcoreMesh(**_SC_MESH),
        scratch_types=[
            pltpu.VMEM((NCHUNK, CS), jnp.int32),
            pltpu.VMEM((MCS, H), jnp.float32),
            pltpu.VMEM((MCS, H), jnp.float32),
            pltpu.VMEM_SHARED((N, H), jnp.float32),
            pltpu.SemaphoreType.DMA,
            pltpu.SemaphoreType.DMA,
        ],
    )


def _out_body(z_ref, p0_ref, p1_ref, atab_ref, wcat_ref, wcbt_ref, bc_ref, out_ref):
    oh = (z_ref[...] == lax.broadcasted_iota(jnp.int32, (1, NCLS), 1)).astype(
        jnp.float32
    )
    t1 = jnp.dot(atab_ref[...], wcat_ref[...])
    agg = p0_ref[...] + p1_ref[...]
    out_ref[...] = (
        jnp.dot(oh, t1) + jnp.dot(agg, wcbt_ref[...]) + bc_ref[...]
    )


def _out_call(z2, p0, p1, atab_pad, wcat, wcbt, bc2, block_n):
    grid = (N // block_n,)
    return pl.pallas_call(
        _out_body,
        grid=grid,
        in_specs=[
            pl.BlockSpec((block_n, 1), lambda i: (i, 0)),
            pl.BlockSpec((block_n, H), lambda i: (i, 0)),
            pl.BlockSpec((block_n, H), lambda i: (i, 0)),
            pl.BlockSpec((NCLS, H), lambda i: (0, 0)),
            pl.BlockSpec((H, H), lambda i: (0, 0)),
            pl.BlockSpec((H, H), lambda i: (0, 0)),
            pl.BlockSpec((1, H), lambda i: (0, 0)),
        ],
        out_specs=pl.BlockSpec((block_n, H), lambda i: (i, 0)),
        out_shape=jax.ShapeDtypeStruct((N, H), jnp.float32),
    )(z2, p0, p1, atab_pad, wcat, wcbt, bc2)


def kernel(z, edge_index, edge_dist, edge_attr, atom_table, neigh_table, Wd, bd, Wc, bc):
    z = z.astype(jnp.int32)
    row = edge_index[0].astype(jnp.int32)
    col = edge_index[1].astype(jnp.int32)

    zc = _zc_kernel()(z, col.reshape(NW, EPW)).reshape(E)

    wdt = Wd.T
    bd2 = bd.reshape(1, H)
    ntab_pad = jnp.zeros((NCLS, H), jnp.float32).at[: neigh_table.shape[0]].set(neigh_table)
    msg = _msg_call(edge_attr.T, edge_dist, zc, wdt, bd2, ntab_pad, block_e=2560)

    zinit = jnp.zeros((N, H), jnp.float32)
    parts = _scatter_kernel()(msg, row.reshape(NW, NCHUNK, CS), zinit)

    z2 = z.reshape(N, 1)
    atab_pad = jnp.zeros((NCLS, H), jnp.float32).at[: atom_table.shape[0]].set(atom_table)
    wcat = Wc[:, :H].T
    wcbt = Wc[:, H:].T
    bc2 = bc.reshape(1, H)
    return _out_call(z2, parts[0], parts[1], atab_pad, wcat, wcbt, bc2, block_n=1000)


# speedup vs baseline: 5.9483x; 1.1203x over previous
import functools

import jax
import jax.numpy as jnp
from jax import lax
from jax.experimental import pallas as pl
from jax.experimental.pallas import tpu as pltpu
from jax.experimental.pallas import tpu_sc as plsc

N = 10000
E = 320000
H = 128
R = 16
CUTOFF = 5.0
NCLS = 128

NW = 32
CS = 80
BLK_E = 2560
E0 = 153600
E1 = E - E0
NPW = 624
NPW_LAST = N - 15 * NPW

_SC_MESH = dict(core_axis_name="c", subcore_axis_name="s")


ZG = 5


def _make_zc_body(epw):
    nchunk = epw // CS
    ngrp = nchunk // ZG

    def body(z_hbm, col2_hbm, zc_hbm, col_v, zc_v, sem):
        wid = lax.axis_index("s") * 2 + lax.axis_index("c")

        pltpu.sync_copy(col2_hbm.at[wid], col_v)

        def fire(g):
            for j in range(ZG):
                sl = pl.ds((g * ZG + j) * CS, CS)
                pltpu.async_copy(z_hbm.at[col_v.at[sl]], zc_v.at[sl], sem)

        def drain():
            for j in range(ZG):
                pltpu.make_async_copy(
                    z_hbm.at[col_v.at[pl.ds(0, CS)]], zc_v.at[pl.ds(0, CS)], sem
                ).wait()

        fire(0)

        def group(g, carry):
            @pl.when(g + 1 < ngrp)
            def _():
                fire(g + 1)

            drain()
            return carry

        lax.fori_loop(0, ngrp, group, 0)
        pltpu.sync_copy(zc_v, zc_hbm.at[wid])

    return body


@functools.cache
def _zc_kernel(epw):
    return pl.kernel(
        _make_zc_body(epw),
        out_type=jax.ShapeDtypeStruct((NW, epw), jnp.int32),
        mesh=plsc.VectorSubcoreMesh(**_SC_MESH),
        scratch_types=[
            pltpu.VMEM((epw,), jnp.int32),
            pltpu.VMEM((epw,), jnp.int32),
            pltpu.SemaphoreType.DMA,
        ],
    )


MB_GRP = 512


def _msg_body(eat_ref, dist_ref, zc_ref, wdt_ref, bd_ref, ntab_ref, msg_ref):
    be = eat_ref.shape[1]
    for j in range(be // MB_GRP):
        sl = pl.ds(j * MB_GRP, MB_GRP)
        d = dist_ref[0, 0, sl]
        c = 0.5 * (jnp.cos(jnp.pi * d / CUTOFF) + 1.0)
        c = c * (d < CUTOFF).astype(jnp.float32)
        w = (
            lax.dot_general(
                eat_ref[:, sl], wdt_ref[...], (((0,), (0,)), ((), ()))
            )
            + bd_ref[...]
        )
        ohct = (
            lax.broadcasted_iota(jnp.int32, (NCLS, MB_GRP), 0)
            == zc_ref[0, 0, sl][None, :]
        ).astype(jnp.float32) * c[None, :]
        nfc = lax.dot_general(ohct, ntab_ref[...], (((0,), (0,)), ((), ())))
        msg_ref[sl, :] = w * nfc


def _msg_call(eat, dist3, zc, wdt, bd2, ntab_pad, nedge, blk0):
    nblk = nedge // BLK_E
    zc3 = zc.reshape(nblk, 1, BLK_E)
    return pl.pallas_call(
        _msg_body,
        grid=(nblk,),
        in_specs=[
            pl.BlockSpec((R, BLK_E), lambda i: (0, i + blk0)),
            pl.BlockSpec((1, 1, BLK_E), lambda i: (i + blk0, 0, 0)),
            pl.BlockSpec((1, 1, BLK_E), lambda i: (i, 0, 0)),
            pl.BlockSpec((R, H), lambda i: (0, 0)),
            pl.BlockSpec((1, H), lambda i: (0, 0)),
            pl.BlockSpec((NCLS, H), lambda i: (0, 0)),
        ],
        out_specs=pl.BlockSpec((BLK_E, H), lambda i: (i, 0)),
        out_shape=jax.ShapeDtypeStruct((nedge, H), jnp.float32),
        compiler_params=pltpu.CompilerParams(
            dimension_semantics=("parallel",),
        ),
    )(eat, dist3, zc3, wdt, bd2, ntab_pad)


MCS = CS


def _make_scatter_body(epw):
    nmc = epw // MCS

    def body(
        msg_hbm, row3_hbm, zinit_hbm, parts_hbm,
        row_v, msg_a, msg_b, agg_sp, sem_a, sem_b,
    ):
        cid = lax.axis_index("c")
        sid = lax.axis_index("s")
        wid = sid * 2 + cid
        base = wid * epw

        nbase = sid * NPW

        @pl.when(sid < 15)
        def _():
            pltpu.sync_copy(
                zinit_hbm.at[pl.ds(nbase, NPW)], agg_sp.at[pl.ds(nbase, NPW)]
            )

        @pl.when(sid == 15)
        def _():
            pltpu.sync_copy(
                zinit_hbm.at[pl.ds(nbase, NPW_LAST)],
                agg_sp.at[pl.ds(nbase, NPW_LAST)],
            )

        pltpu.sync_copy(row3_hbm.at[wid], row_v)
        plsc.subcore_barrier()

        pltpu.async_copy(msg_hbm.at[pl.ds(base, MCS)], msg_a, sem_a)

        def scatter_from(buf, t):
            pltpu.sync_copy(buf, agg_sp.at[row_v.at[t]], add=True)

        def pair(p, carry):
            t0 = p * 2
            pltpu.make_async_copy(
                msg_hbm.at[pl.ds(base, MCS)], msg_a, sem_a
            ).wait()

            @pl.when(t0 + 1 < nmc)
            def _():
                pltpu.async_copy(
                    msg_hbm.at[pl.ds(base + (t0 + 1) * MCS, MCS)], msg_b, sem_b
                )

            scatter_from(msg_a, t0)

            @pl.when(t0 + 1 < nmc)
            def _():
                pltpu.make_async_copy(
                    msg_hbm.at[pl.ds(base, MCS)], msg_b, sem_b
                ).wait()

                @pl.when(t0 + 2 < nmc)
                def _():
                    pltpu.async_copy(
                        msg_hbm.at[pl.ds(base + (t0 + 2) * MCS, MCS)],
                        msg_a,
                        sem_a,
                    )

                scatter_from(msg_b, t0 + 1)

            return carry

        lax.fori_loop(0, (nmc + 1) // 2, pair, 0)
        plsc.subcore_barrier()

        @pl.when(sid < 15)
        def _():
            pltpu.sync_copy(
                agg_sp.at[pl.ds(nbase, NPW)], parts_hbm.at[cid, pl.ds(nbase, NPW)]
            )

        @pl.when(sid == 15)
        def _():
            pltpu.sync_copy(
                agg_sp.at[pl.ds(nbase, NPW_LAST)],
                parts_hbm.at[cid, pl.ds(nbase, NPW_LAST)],
            )

    return body


@functools.cache
def _scatter_kernel(epw):
    return pl.kernel(
        _make_scatter_body(epw),
        out_type=jax.ShapeDtypeStruct((2, N, H), jnp.float32),
        mesh=plsc.VectorSubcoreMesh(**_SC_MESH),
        scratch_types=[
            pltpu.VMEM((epw // MCS, CS), jnp.int32),
            pltpu.VMEM((MCS, H), jnp.float32),
            pltpu.VMEM((MCS, H), jnp.float32),
            pltpu.VMEM_SHARED((N, H), jnp.float32),
            pltpu.SemaphoreType.DMA,
            pltpu.SemaphoreType.DMA,
        ],
    )


def _out_body(
    z_ref, p0_ref, p1_ref, p2_ref, p3_ref, atab_ref, wcat_ref, wcbt_ref, bc_ref,
    out_ref,
):
    oh = (z_ref[...] == lax.broadcasted_iota(jnp.int32, (1, NCLS), 1)).astype(
        jnp.float32
    )
    t1 = jnp.dot(atab_ref[...], wcat_ref[...])
    agg = (p0_ref[...] + p1_ref[...]) + (p2_ref[...] + p3_ref[...])
    out_ref[...] = (
        jnp.dot(oh, t1) + jnp.dot(agg, wcbt_ref[...]) + bc_ref[...]
    )


def _out_call(z2, parts0, parts1, atab_pad, wcat, wcbt, bc2, block_n):
    grid = (N // block_n,)
    return pl.pallas_call(
        _out_body,
        grid=grid,
        in_specs=[
            pl.BlockSpec((block_n, 1), lambda i: (i, 0)),
            pl.BlockSpec((block_n, H), lambda i: (i, 0)),
            pl.BlockSpec((block_n, H), lambda i: (i, 0)),
            pl.BlockSpec((block_n, H), lambda i: (i, 0)),
            pl.BlockSpec((block_n, H), lambda i: (i, 0)),
            pl.BlockSpec((NCLS, H), lambda i: (0, 0)),
            pl.BlockSpec((H, H), lambda i: (0, 0)),
            pl.BlockSpec((H, H), lambda i: (0, 0)),
            pl.BlockSpec((1, H), lambda i: (0, 0)),
        ],
        out_specs=pl.BlockSpec((block_n, H), lambda i: (i, 0)),
        out_shape=jax.ShapeDtypeStruct((N, H), jnp.float32),
    )(
        z2, parts0[0], parts0[1], parts1[0], parts1[1],
        atab_pad, wcat, wcbt, bc2,
    )


def kernel(z, edge_index, edge_dist, edge_attr, atom_table, neigh_table, Wd, bd, Wc, bc):
    z = z.astype(jnp.int32)
    row = edge_index[0].astype(jnp.int32)
    col = edge_index[1].astype(jnp.int32)

    zc0 = _zc_kernel(E0 // NW)(z, col[:E0].reshape(NW, E0 // NW)).reshape(E0)
    zc1 = _zc_kernel(E1 // NW)(z, col[E0:].reshape(NW, E1 // NW)).reshape(E1)

    wdt = Wd.T
    bd2 = bd.reshape(1, H)
    ntab_pad = jnp.zeros((NCLS, H), jnp.float32).at[: neigh_table.shape[0]].set(neigh_table)
    eat = edge_attr.T
    dist3 = edge_dist.reshape(E // BLK_E, 1, BLK_E)
    zinit = jnp.zeros((N, H), jnp.float32)

    msg0 = _msg_call(eat, dist3, zc0, wdt, bd2, ntab_pad, E0, 0)
    parts0 = _scatter_kernel(E0 // NW)(
        msg0, row[:E0].reshape(NW, E0 // NW // CS, CS), zinit
    )
    msg1 = _msg_call(eat, dist3, zc1, wdt, bd2, ntab_pad, E1, E0 // BLK_E)
    parts1 = _scatter_kernel(E1 // NW)(
        msg1, row[E0:].reshape(NW, E1 // NW // CS, CS), zinit
    )

    z2 = z.reshape(N, 1)
    atab_pad = jnp.zeros((NCLS, H), jnp.float32).at[: atom_table.shape[0]].set(atom_table)
    wcat = Wc[:, :H].T
    wcbt = Wc[:, H:].T
    bc2 = bc.reshape(1, H)
    return _out_call(z2, parts0, parts1, atab_pad, wcat, wcbt, bc2, block_n=1000)
